# baseline (speedup 1.0000x reference)
import functools

import jax
import jax.numpy as jnp
from jax import lax
from jax.experimental import pallas as pl
from jax.experimental.pallas import tpu as pltpu
from jax.experimental.pallas import tpu_sc as plsc

NUM_WORKERS = 32
LANE = 128
EMB_DIM = 32
RING = 8


def _build(batch: int, seq: int, dim: int):
    b_per_w = batch // NUM_WORKERS
    blocks_per_t = b_per_w // LANE
    sub = dim // 8
    n_bhi = batch // LANE

    mesh = plsc.VectorSubcoreMesh(core_axis_name="c", subcore_axis_name="s")

    @functools.partial(
        pl.kernel,
        mesh=mesh,
        compiler_params=pltpu.CompilerParams(
            use_tc_tiling_on_sc=False, needs_layout_passes=False
        ),
        out_type=jax.ShapeDtypeStruct((seq, sub, n_bhi, 8, LANE), jnp.float32),
        scratch_types=[
            pltpu.VMEM((seq, b_per_w), jnp.int32),
            pltpu.VMEM((RING, LANE, dim), jnp.float32),
            pltpu.VMEM((RING, sub, 8, LANE), jnp.float32),
        ]
        + [pltpu.SemaphoreType.DMA] * (2 * RING),
    )
    def k(table_hbm, idx_hbm, out_hbm, idx_v, rv, tv, *sems):
        gs = sems[:RING]
        os_ = sems[RING:]
        wid = lax.axis_index("c") * 16 + lax.axis_index("s")
        pltpu.sync_copy(idx_hbm.at[:, pl.ds(wid * b_per_w, b_per_w)], idx_v)

        iota16 = lax.iota(jnp.int32, 16)
        chi_lo = jnp.right_shift(iota16, 3)
        chi_hi = chi_lo + 2
        clo = jnp.bitwise_and(iota16, 7)

        def fire(t_idx, bl, p):
            ivec = idx_v.at[t_idx, pl.ds(bl * LANE, LANE)]
            pltpu.async_copy(table_hbm.at[ivec], rv.at[p], gs[p])

        def wait_g(p):
            pltpu.make_async_copy(
                table_hbm.at[idx_v.at[0, pl.ds(0, LANE)]], rv.at[p], gs[p]
            ).wait()

        def transpose(p):
            src = rv.at[p]
            dst = tv.at[p]

            @plsc.parallel_loop(0, LANE, 1, unroll=16)
            def _(b):
                cols = jnp.full((16,), 0, jnp.int32) + b
                v_lo = src[b, pl.ds(0, 16)]
                v_hi = src[b, pl.ds(16, 16)]
                plsc.store_scatter(dst, [chi_lo, clo, cols], v_lo)
                plsc.store_scatter(dst, [chi_hi, clo, cols], v_hi)

        def fire_writes(t_idx, bl, p):
            pltpu.async_copy(
                tv.at[p],
                out_hbm.at[t_idx, pl.ds(0, sub), wid * blocks_per_t + bl],
                os_[p],
            )

        def wait_writes(p):
            pltpu.make_async_copy(
                tv.at[p],
                out_hbm.at[0, pl.ds(0, sub), 0],
                os_[p],
            ).wait()

        for p in range(RING):
            fire(p // blocks_per_t, p % blocks_per_t, p)

        def ubody(u, carry):
            t0 = 2 * u
            for p in range(RING):
                t = t0 + p // blocks_per_t
                bl = p % blocks_per_t
                t_next = jnp.minimum(t + 2, seq - 1)
                wait_g(p)

                @pl.when(u > 0)
                def _():
                    wait_writes(p)

                transpose(p)
                fire(t_next, bl, p)
                fire_writes(t, bl, p)
            return carry

        lax.fori_loop(0, seq // 2, ubody, 0)
        for p in range(RING):
            wait_g(p)
            wait_writes(p)

    return k


def kernel(inputs, weight):
    b, t = inputs.shape
    idxT = inputs.T.astype(jnp.int32)
    k = _build(b, t, EMB_DIM)
    out5 = k(weight, idxT)
    return out5.transpose(2, 4, 0, 1, 3).reshape(b, t, EMB_DIM)
